# Initial kernel scaffold; baseline (speedup 1.0000x reference)
import jax
import jax.numpy as jnp
from jax.experimental import pallas as pl


def kernel(x, edge_index, W0_l, W0_r, b0, W1_l, W1_r, b1, W2_l, W2_r, b2, Ws1, bs1, Ws2, bs2, Wn1, bn1, Wn2, bn2):
    raise NotImplementedError("write your pallas kernel here")


# speedup vs baseline: 3.2411x; 3.2411x over previous
import jax
import jax.numpy as jnp
from jax import lax
from jax.experimental import pallas as pl
from jax.experimental.pallas import tpu as pltpu
from jax.experimental.pallas import tpu_sc as plsc

_N = 10000
_H = 256
_TILES = 16
_B = 128
_EPT = 20000
_CH = 160
_EPT0 = 10000
_CH0 = 80
_G = 8
_NPAD = 10112
_RPT = _NPAD // _TILES


def _make_sc_agg0():
    f32 = jnp.float32
    outs = (jax.ShapeDtypeStruct((_NPAD, 128), f32),
            jax.ShapeDtypeStruct((_NPAD, 128), f32),
            jax.ShapeDtypeStruct((2 * _TILES, _RPT), f32))
    scratch = [
        pltpu.VMEM((_G, _B), jnp.int32),
        pltpu.VMEM((_G, _B), jnp.int32),
        pltpu.VMEM((_B, 128), f32),
        pltpu.SemaphoreType.DMA,
        pltpu.VMEM_SHARED((_NPAD, 128), f32),
        pltpu.VMEM_SHARED((_NPAD,), f32),
        pltpu.VMEM((_B,), f32),
        pltpu.VMEM((_RPT,), f32),
    ]
    mesh = plsc.VectorSubcoreMesh(core_axis_name="c", subcore_axis_name="s")

    def body(z2, z1, srcs, dsts, xf, p0, p1, degp,
             src_v, dst_v, rows_v, sem, acc, deg_acc, ones_v, dz_v):
        c = lax.axis_index("c")
        t = lax.axis_index("s")
        wid = c * _TILES + t
        r0 = t * _RPT
        pltpu.sync_copy(z2.at[pl.ds(r0, _RPT)], acc.at[pl.ds(r0, _RPT)])
        pltpu.sync_copy(z1.at[pl.ds(r0, _RPT)], dz_v)
        pltpu.sync_copy(dz_v, deg_acc.at[pl.ds(r0, _RPT)])
        for k in range(_B // 16):
            ones_v[pl.ds(k * 16, 16)] = jnp.ones((16,), f32)
        plsc.subcore_barrier()

        def group(g, carry):
            pltpu.sync_copy(srcs.at[wid, pl.ds(g * _G, _G)], src_v)
            pltpu.sync_copy(dsts.at[wid, pl.ds(g * _G, _G)], dst_v)

            def chunk(j, carry2):
                pltpu.async_copy(xf.at[src_v.at[j]], rows_v, sem).wait()
                pltpu.sync_copy(rows_v, acc.at[dst_v.at[j]], add=True)
                pltpu.sync_copy(ones_v, deg_acc.at[dst_v.at[j]], add=True)
                return carry2
            lax.fori_loop(0, _G, chunk, 0)
            return carry
        lax.fori_loop(0, _CH0 // _G, group, 0)
        plsc.subcore_barrier()
        for cc in range(2):
            out = (p0, p1)[cc]

            @pl.when(c == cc)
            def _():
                pltpu.sync_copy(acc.at[pl.ds(r0, _RPT)],
                                out.at[pl.ds(r0, _RPT)])
        pltpu.sync_copy(deg_acc.at[pl.ds(r0, _RPT)], dz_v)
        pltpu.sync_copy(dz_v, degp.at[wid])

    return pl.kernel(body, out_type=outs, mesh=mesh, scratch_types=scratch)


def _make_sc_agg():
    f32 = jnp.float32
    outs = (jax.ShapeDtypeStruct((_NPAD, 128), f32),
            jax.ShapeDtypeStruct((_NPAD, 128), f32))
    scratch = [
        pltpu.VMEM((_G, _B), jnp.int32),
        pltpu.VMEM((_G, _B), jnp.int32),
        pltpu.VMEM((_B, 128), f32),
        pltpu.SemaphoreType.DMA,
        pltpu.VMEM_SHARED((_NPAD, 128), f32),
    ]
    mesh = plsc.VectorSubcoreMesh(core_axis_name="c", subcore_axis_name="s")

    def body(z2, srcs, dsts, t0, t1, a0, a1,
             src_v, dst_v, rows_v, sem, acc):
        c = lax.axis_index("c")
        t = lax.axis_index("s")
        r0 = t * _RPT
        pltpu.sync_copy(z2.at[pl.ds(r0, _RPT)], acc.at[pl.ds(r0, _RPT)])
        plsc.subcore_barrier()
        for cc in range(2):
            tbl = (t0, t1)[cc]
            out = (a0, a1)[cc]

            @pl.when(c == cc)
            def _():
                def group(g, carry):
                    pltpu.sync_copy(srcs.at[t, pl.ds(g * _G, _G)], src_v)
                    pltpu.sync_copy(dsts.at[t, pl.ds(g * _G, _G)], dst_v)

                    def chunk(j, carry2):
                        pltpu.async_copy(tbl.at[src_v.at[j]], rows_v, sem).wait()
                        pltpu.sync_copy(rows_v, acc.at[dst_v.at[j]], add=True)
                        return carry2
                    lax.fori_loop(0, _G, chunk, 0)
                    return carry
                lax.fori_loop(0, _CH // _G, group, 0)
                plsc.subcore_barrier()
                pltpu.sync_copy(acc.at[pl.ds(r0, _RPT)],
                                out.at[pl.ds(r0, _RPT)])

    return pl.kernel(body, out_type=outs, mesh=mesh, scratch_types=scratch)


_BN = 1000
_full = lambda shape: pl.BlockSpec(shape, lambda i: (0, 0))
_rows = lambda w: pl.BlockSpec((_BN, w), lambda i: (i, 0))


def _make_tc_layer0():
    f32 = jnp.float32
    hh = _H // 2

    def body(p0, p1, xf, d0, d1, wl, wr, b, o0, o1, sc_o):
        scale = 1.0 / jnp.maximum(d0[...] + d1[...], 1.0)
        acc = (jnp.dot((p0[...] + p1[...]) * scale, wl[...],
                       preferred_element_type=f32)
               + jnp.dot(xf[...], wr[...], preferred_element_type=f32)
               + b[...])
        acc = jnp.maximum(acc, 0.0)
        o0[...] = acc[:, :hh]
        o1[...] = acc[:, hh:]
        sc_o[...] = scale

    return pl.pallas_call(
        body,
        grid=(_N // _BN,),
        in_specs=[_rows(128), _rows(128), _rows(128), _rows(1), _rows(1),
                  _full((128, _H)), _full((128, _H)), _full((1, _H))],
        out_specs=[_rows(hh), _rows(hh), _rows(1)],
        out_shape=[jax.ShapeDtypeStruct((_N, hh), f32),
                   jax.ShapeDtypeStruct((_N, hh), f32),
                   jax.ShapeDtypeStruct((_N, 1), f32)],
    )


def _make_tc_layer(relu):
    f32 = jnp.float32
    dh = 128
    hh = _H // 2

    def body(a0, a1, h0, h1, sc, wl, wr, b, o0, o1):
        scale = sc[...]
        acc = (jnp.dot(a0[...] * scale, wl[0:dh, :], preferred_element_type=f32)
               + jnp.dot(a1[...] * scale, wl[dh:2 * dh, :], preferred_element_type=f32)
               + jnp.dot(h0[...], wr[0:dh, :], preferred_element_type=f32)
               + jnp.dot(h1[...], wr[dh:2 * dh, :], preferred_element_type=f32)
               + b[...])
        if relu:
            acc = jnp.maximum(acc, 0.0)
        o0[...] = acc[:, :hh]
        o1[...] = acc[:, hh:]

    return pl.pallas_call(
        body,
        grid=(_N // _BN,),
        in_specs=[_rows(dh), _rows(dh), _rows(dh), _rows(dh), _rows(1),
                  _full((2 * dh, _H)), _full((2 * dh, _H)), _full((1, _H))],
        out_specs=[_rows(hh), _rows(hh)],
        out_shape=[jax.ShapeDtypeStruct((_N, hh), f32),
                   jax.ShapeDtypeStruct((_N, hh), f32)],
    )


def _make_tc_final():
    f32 = jnp.float32
    dh = 128

    def body(a0, a1, h0, h1, sc, wl, wr, b,
             ws1, bs1, ws2, bs2, wn1, bn1, wn2, bn2, seg_o, nud_o):
        scale = sc[...]
        h = (jnp.dot(a0[...] * scale, wl[0:dh, :], preferred_element_type=f32)
             + jnp.dot(a1[...] * scale, wl[dh:2 * dh, :], preferred_element_type=f32)
             + jnp.dot(h0[...], wr[0:dh, :], preferred_element_type=f32)
             + jnp.dot(h1[...], wr[dh:2 * dh, :], preferred_element_type=f32)
             + b[...])
        sh = jnp.maximum(jnp.dot(h, ws1[...], preferred_element_type=f32)
                         + bs1[...], 0.0)
        seg_o[...] = jnp.dot(sh, ws2[...], preferred_element_type=f32) + bs2[...]
        nh = jnp.maximum(jnp.dot(h, wn1[...], preferred_element_type=f32)
                         + bn1[...], 0.0)
        nud_o[...] = jnp.dot(nh, wn2[...], preferred_element_type=f32) + bn2[...]

    return pl.pallas_call(
        body,
        grid=(_N // _BN,),
        in_specs=[_rows(dh), _rows(dh), _rows(dh), _rows(dh), _rows(1),
                  _full((2 * dh, _H)), _full((2 * dh, _H)), _full((1, _H)),
                  _full((_H, _H)), _full((1, _H)), _full((_H, 6)), _full((1, 6)),
                  _full((_H, _H)), _full((1, _H)), _full((_H, 5)), _full((1, 5))],
        out_specs=[_rows(6), _rows(5)],
        out_shape=[jax.ShapeDtypeStruct((_N, 6), f32),
                   jax.ShapeDtypeStruct((_N, 5), f32)],
    )


def kernel(x, edge_index, W0_l, W0_r, b0, W1_l, W1_r, b1, W2_l, W2_r, b2,
           Ws1, bs1, Ws2, bs2, Wn1, bn1, Wn2, bn2):
    f32 = jnp.float32
    src = edge_index[0]
    dst = edge_index[1]
    srcs0 = jnp.pad(src.reshape(2 * _TILES, _EPT0),
                    ((0, 0), (0, _CH0 * _B - _EPT0))).reshape(2 * _TILES, _CH0, _B)
    dsts0 = jnp.pad(dst.reshape(2 * _TILES, _EPT0),
                    ((0, 0), (0, _CH0 * _B - _EPT0)),
                    constant_values=_N).reshape(2 * _TILES, _CH0, _B)
    srcs = jnp.pad(src.reshape(_TILES, _EPT),
                   ((0, 0), (0, _CH * _B - _EPT))).reshape(_TILES, _CH, _B)
    dsts = jnp.pad(dst.reshape(_TILES, _EPT),
                   ((0, 0), (0, _CH * _B - _EPT)),
                   constant_values=_N).reshape(_TILES, _CH, _B)
    z128 = jnp.zeros((_NPAD, 128), f32)
    z1d = jnp.zeros((_NPAD,), f32)

    p0, p1, degp = _make_sc_agg0()(z128, z1d, srcs0, dsts0, x)
    d0 = degp[:_TILES].reshape(-1)[:_N].reshape(_N, 1)
    d1 = degp[_TILES:].reshape(-1)[:_N].reshape(_N, 1)

    h0, h1, scale = _make_tc_layer0()(
        p0[:_N], p1[:_N], x, d0, d1, W0_l, W0_r, b0.reshape(1, -1))

    sc_mid = _make_sc_agg()
    a0p, a1p = sc_mid(z128, srcs, dsts, h0, h1)
    h0, h1 = _make_tc_layer(True)(
        a0p[:_N], a1p[:_N], h0, h1, scale, W1_l, W1_r, b1.reshape(1, -1))

    a0p, a1p = sc_mid(z128, srcs, dsts, h0, h1)
    seg, nud = _make_tc_final()(
        a0p[:_N], a1p[:_N], h0, h1, scale, W2_l, W2_r, b2.reshape(1, -1),
        Ws1, bs1.reshape(1, -1), Ws2, bs2.reshape(1, -1),
        Wn1, bn1.reshape(1, -1), Wn2, bn2.reshape(1, -1))
    return (seg, nud)


# speedup vs baseline: 3.7726x; 1.1640x over previous
import jax
import jax.numpy as jnp
from jax import lax
from jax.experimental import pallas as pl
from jax.experimental.pallas import tpu as pltpu
from jax.experimental.pallas import tpu_sc as plsc

_N = 10000
_H = 256
_TILES = 16
_B = 128
_EPT = 20000
_CH = 160
_EPT0 = 10000
_CH0 = 80
_G = 8
_NPAD = 10112
_RPT = _NPAD // _TILES


def _make_sc_agg0():
    f32 = jnp.float32
    outs = (jax.ShapeDtypeStruct((_NPAD, 128), f32),
            jax.ShapeDtypeStruct((_NPAD, 128), f32),
            jax.ShapeDtypeStruct((2 * _TILES, _RPT), f32))
    scratch = [
        pltpu.VMEM((_G, _B), jnp.int32),
        pltpu.VMEM((_G, _B), jnp.int32),
        pltpu.VMEM((_B, 128), f32),
        pltpu.VMEM((_B, 128), f32),
        pltpu.SemaphoreType.DMA,
        pltpu.SemaphoreType.DMA,
        pltpu.VMEM_SHARED((_NPAD, 128), f32),
        pltpu.VMEM_SHARED((_NPAD,), f32),
        pltpu.VMEM((_B,), f32),
        pltpu.VMEM((_RPT,), f32),
    ]
    mesh = plsc.VectorSubcoreMesh(core_axis_name="c", subcore_axis_name="s")

    def body(z2, z1, srcs, dsts, xf, p0, p1, degp,
             src_v, dst_v, rows_a, rows_b, sem_a, sem_b,
             acc, deg_acc, ones_v, dz_v):
        c = lax.axis_index("c")
        t = lax.axis_index("s")
        wid = c * _TILES + t
        r0 = t * _RPT
        pltpu.sync_copy(z2.at[pl.ds(r0, _RPT)], acc.at[pl.ds(r0, _RPT)])
        pltpu.sync_copy(z1.at[pl.ds(r0, _RPT)], dz_v)
        pltpu.sync_copy(dz_v, deg_acc.at[pl.ds(r0, _RPT)])
        for k in range(_B // 16):
            ones_v[pl.ds(k * 16, 16)] = jnp.ones((16,), f32)
        plsc.subcore_barrier()

        bufs = ((rows_a, sem_a), (rows_b, sem_b))

        def group(g, carry):
            pltpu.sync_copy(srcs.at[wid, pl.ds(g * _G, _G)], src_v)
            pltpu.sync_copy(dsts.at[wid, pl.ds(g * _G, _G)], dst_v)
            pend = [pltpu.async_copy(xf.at[src_v.at[0]], bufs[0][0],
                                     bufs[0][1])]
            for k in range(_G):
                buf, _ = bufs[k % 2]
                if k + 1 < _G:
                    nbuf, nsem = bufs[(k + 1) % 2]
                    pend.append(pltpu.async_copy(xf.at[src_v.at[k + 1]],
                                                 nbuf, nsem))
                pend.pop(0).wait()
                pltpu.sync_copy(buf, acc.at[dst_v.at[k]], add=True)
                pltpu.sync_copy(ones_v, deg_acc.at[dst_v.at[k]], add=True)
            return carry
        lax.fori_loop(0, _CH0 // _G, group, 0)
        plsc.subcore_barrier()
        for cc in range(2):
            out = (p0, p1)[cc]

            @pl.when(c == cc)
            def _():
                pltpu.sync_copy(acc.at[pl.ds(r0, _RPT)],
                                out.at[pl.ds(r0, _RPT)])
        pltpu.sync_copy(deg_acc.at[pl.ds(r0, _RPT)], dz_v)
        pltpu.sync_copy(dz_v, degp.at[wid])

    return pl.kernel(body, out_type=outs, mesh=mesh, scratch_types=scratch)


def _make_sc_agg():
    f32 = jnp.float32
    outs = (jax.ShapeDtypeStruct((_NPAD, 128), f32),
            jax.ShapeDtypeStruct((_NPAD, 128), f32))
    scratch = [
        pltpu.VMEM((_G, _B), jnp.int32),
        pltpu.VMEM((_G, _B), jnp.int32),
        pltpu.VMEM((_B, 128), f32),
        pltpu.VMEM((_B, 128), f32),
        pltpu.SemaphoreType.DMA,
        pltpu.SemaphoreType.DMA,
        pltpu.VMEM_SHARED((_NPAD, 128), f32),
    ]
    mesh = plsc.VectorSubcoreMesh(core_axis_name="c", subcore_axis_name="s")

    def body(z2, srcs, dsts, t0, t1, a0, a1,
             src_v, dst_v, rows_a, rows_b, sem_a, sem_b, acc):
        c = lax.axis_index("c")
        t = lax.axis_index("s")
        r0 = t * _RPT
        pltpu.sync_copy(z2.at[pl.ds(r0, _RPT)], acc.at[pl.ds(r0, _RPT)])
        plsc.subcore_barrier()
        for cc in range(2):
            tbl = (t0, t1)[cc]
            out = (a0, a1)[cc]

            @pl.when(c == cc)
            def _():
                bufs = ((rows_a, sem_a), (rows_b, sem_b))

                def group(g, carry):
                    pltpu.sync_copy(srcs.at[t, pl.ds(g * _G, _G)], src_v)
                    pltpu.sync_copy(dsts.at[t, pl.ds(g * _G, _G)], dst_v)
                    pend = [pltpu.async_copy(tbl.at[src_v.at[0]],
                                             bufs[0][0], bufs[0][1])]
                    for k in range(_G):
                        buf, _ = bufs[k % 2]
                        if k + 1 < _G:
                            nbuf, nsem = bufs[(k + 1) % 2]
                            pend.append(pltpu.async_copy(
                                tbl.at[src_v.at[k + 1]], nbuf, nsem))
                        pend.pop(0).wait()
                        pltpu.sync_copy(buf, acc.at[dst_v.at[k]], add=True)
                    return carry
                lax.fori_loop(0, _CH // _G, group, 0)
                plsc.subcore_barrier()
                pltpu.sync_copy(acc.at[pl.ds(r0, _RPT)],
                                out.at[pl.ds(r0, _RPT)])

    return pl.kernel(body, out_type=outs, mesh=mesh, scratch_types=scratch)


_BN = 1000
_full = lambda shape: pl.BlockSpec(shape, lambda i: (0, 0))
_rows = lambda w: pl.BlockSpec((_BN, w), lambda i: (i, 0))


def _make_tc_layer0():
    f32 = jnp.float32
    hh = _H // 2

    def body(p0, p1, xf, d0, d1, wl, wr, b, o0, o1, sc_o):
        scale = 1.0 / jnp.maximum(d0[...] + d1[...], 1.0)
        acc = (jnp.dot((p0[...] + p1[...]) * scale, wl[...],
                       preferred_element_type=f32)
               + jnp.dot(xf[...], wr[...], preferred_element_type=f32)
               + b[...])
        acc = jnp.maximum(acc, 0.0)
        o0[...] = acc[:, :hh]
        o1[...] = acc[:, hh:]
        sc_o[...] = scale

    return pl.pallas_call(
        body,
        grid=(_N // _BN,),
        in_specs=[_rows(128), _rows(128), _rows(128), _rows(1), _rows(1),
                  _full((128, _H)), _full((128, _H)), _full((1, _H))],
        out_specs=[_rows(hh), _rows(hh), _rows(1)],
        out_shape=[jax.ShapeDtypeStruct((_N, hh), f32),
                   jax.ShapeDtypeStruct((_N, hh), f32),
                   jax.ShapeDtypeStruct((_N, 1), f32)],
    )


def _make_tc_layer(relu):
    f32 = jnp.float32
    dh = 128
    hh = _H // 2

    def body(a0, a1, h0, h1, sc, wl, wr, b, o0, o1):
        scale = sc[...]
        acc = (jnp.dot(a0[...] * scale, wl[0:dh, :], preferred_element_type=f32)
               + jnp.dot(a1[...] * scale, wl[dh:2 * dh, :], preferred_element_type=f32)
               + jnp.dot(h0[...], wr[0:dh, :], preferred_element_type=f32)
               + jnp.dot(h1[...], wr[dh:2 * dh, :], preferred_element_type=f32)
               + b[...])
        if relu:
            acc = jnp.maximum(acc, 0.0)
        o0[...] = acc[:, :hh]
        o1[...] = acc[:, hh:]

    return pl.pallas_call(
        body,
        grid=(_N // _BN,),
        in_specs=[_rows(dh), _rows(dh), _rows(dh), _rows(dh), _rows(1),
                  _full((2 * dh, _H)), _full((2 * dh, _H)), _full((1, _H))],
        out_specs=[_rows(hh), _rows(hh)],
        out_shape=[jax.ShapeDtypeStruct((_N, hh), f32),
                   jax.ShapeDtypeStruct((_N, hh), f32)],
    )


def _make_tc_final():
    f32 = jnp.float32
    dh = 128

    def body(a0, a1, h0, h1, sc, wl, wr, b,
             ws1, bs1, ws2, bs2, wn1, bn1, wn2, bn2, seg_o, nud_o):
        scale = sc[...]
        h = (jnp.dot(a0[...] * scale, wl[0:dh, :], preferred_element_type=f32)
             + jnp.dot(a1[...] * scale, wl[dh:2 * dh, :], preferred_element_type=f32)
             + jnp.dot(h0[...], wr[0:dh, :], preferred_element_type=f32)
             + jnp.dot(h1[...], wr[dh:2 * dh, :], preferred_element_type=f32)
             + b[...])
        sh = jnp.maximum(jnp.dot(h, ws1[...], preferred_element_type=f32)
                         + bs1[...], 0.0)
        seg_o[...] = jnp.dot(sh, ws2[...], preferred_element_type=f32) + bs2[...]
        nh = jnp.maximum(jnp.dot(h, wn1[...], preferred_element_type=f32)
                         + bn1[...], 0.0)
        nud_o[...] = jnp.dot(nh, wn2[...], preferred_element_type=f32) + bn2[...]

    return pl.pallas_call(
        body,
        grid=(_N // _BN,),
        in_specs=[_rows(dh), _rows(dh), _rows(dh), _rows(dh), _rows(1),
                  _full((2 * dh, _H)), _full((2 * dh, _H)), _full((1, _H)),
                  _full((_H, _H)), _full((1, _H)), _full((_H, 6)), _full((1, 6)),
                  _full((_H, _H)), _full((1, _H)), _full((_H, 5)), _full((1, 5))],
        out_specs=[_rows(6), _rows(5)],
        out_shape=[jax.ShapeDtypeStruct((_N, 6), f32),
                   jax.ShapeDtypeStruct((_N, 5), f32)],
    )


def kernel(x, edge_index, W0_l, W0_r, b0, W1_l, W1_r, b1, W2_l, W2_r, b2,
           Ws1, bs1, Ws2, bs2, Wn1, bn1, Wn2, bn2):
    f32 = jnp.float32
    src = edge_index[0]
    dst = edge_index[1]
    srcs0 = jnp.pad(src.reshape(2 * _TILES, _EPT0),
                    ((0, 0), (0, _CH0 * _B - _EPT0))).reshape(2 * _TILES, _CH0, _B)
    dsts0 = jnp.pad(dst.reshape(2 * _TILES, _EPT0),
                    ((0, 0), (0, _CH0 * _B - _EPT0)),
                    constant_values=_N).reshape(2 * _TILES, _CH0, _B)
    srcs = jnp.pad(src.reshape(_TILES, _EPT),
                   ((0, 0), (0, _CH * _B - _EPT))).reshape(_TILES, _CH, _B)
    dsts = jnp.pad(dst.reshape(_TILES, _EPT),
                   ((0, 0), (0, _CH * _B - _EPT)),
                   constant_values=_N).reshape(_TILES, _CH, _B)
    z128 = jnp.zeros((_NPAD, 128), f32)
    z1d = jnp.zeros((_NPAD,), f32)

    p0, p1, degp = _make_sc_agg0()(z128, z1d, srcs0, dsts0, x)
    d0 = degp[:_TILES].reshape(-1)[:_N].reshape(_N, 1)
    d1 = degp[_TILES:].reshape(-1)[:_N].reshape(_N, 1)

    h0, h1, scale = _make_tc_layer0()(
        p0[:_N], p1[:_N], x, d0, d1, W0_l, W0_r, b0.reshape(1, -1))

    sc_mid = _make_sc_agg()
    a0p, a1p = sc_mid(z128, srcs, dsts, h0, h1)
    h0, h1 = _make_tc_layer(True)(
        a0p[:_N], a1p[:_N], h0, h1, scale, W1_l, W1_r, b1.reshape(1, -1))

    a0p, a1p = sc_mid(z128, srcs, dsts, h0, h1)
    seg, nud = _make_tc_final()(
        a0p[:_N], a1p[:_N], h0, h1, scale, W2_l, W2_r, b2.reshape(1, -1),
        Ws1, bs1.reshape(1, -1), Ws2, bs2.reshape(1, -1),
        Wn1, bn1.reshape(1, -1), Wn2, bn2.reshape(1, -1))
    return (seg, nud)


# speedup vs baseline: 3.9508x; 1.0472x over previous
import jax
import jax.numpy as jnp
from jax import lax
from jax.experimental import pallas as pl
from jax.experimental.pallas import tpu as pltpu
from jax.experimental.pallas import tpu_sc as plsc

_N = 10000
_H = 256
_TILES = 16
_B = 128
_EPT = 20000
_CH = 160
_G = 16
_EPT0 = 10000
_CH0 = 80
_G0 = 8
_NPAD = 10112
_RPT = _NPAD // _TILES


def _emit_group(tbl, acc, ibuf, bufs, gsems, ssems, ng, deg=None):
    gdesc = [None, None]
    sdesc = [None, None]
    ddesc = []
    gdesc[0] = pltpu.async_copy(tbl.at[ibuf.at[0, 0]], bufs[0], gsems[0])
    for k in range(ng):
        p = k % 2
        q = (k + 1) % 2
        if k + 1 < ng:
            if sdesc[q] is not None:
                sdesc[q].wait()
            gdesc[q] = pltpu.async_copy(tbl.at[ibuf.at[k + 1, 0]],
                                        bufs[q], gsems[q])
        gdesc[p].wait()
        sdesc[p] = pltpu.async_copy(bufs[p], acc.at[ibuf.at[k, 1]],
                                    ssems[p], add=True)
        if deg is not None:
            deg_acc, ones_v, dsem = deg
            ddesc.append(pltpu.async_copy(ones_v, deg_acc.at[ibuf.at[k, 1]],
                                          dsem, add=True))
    sdesc[0].wait()
    sdesc[1].wait()
    for d in ddesc:
        d.wait()


def _run_groups(edges_t, tbl, acc, ibufs, isems, bufs, gsems, ssems,
                ng, n_groups, deg=None):
    pltpu.sync_copy(edges_t.at[pl.ds(0, ng)], ibufs[0])

    def pair(m, carry):
        for par in range(2):
            g = 2 * m + par
            cur = ibufs[par]
            nxt = ibufs[1 - par]

            @pl.when(g > 0)
            def _():
                pltpu.make_async_copy(edges_t.at[pl.ds(0, ng)], cur,
                                      isems[par]).wait()

            @pl.when(g + 1 < n_groups)
            def _():
                pltpu.async_copy(edges_t.at[pl.ds((g + 1) * ng, ng)], nxt,
                                 isems[1 - par])
            _emit_group(tbl, acc, cur, bufs, gsems, ssems, ng, deg)
        return carry
    lax.fori_loop(0, n_groups // 2, pair, 0)


def _make_sc_agg0():
    f32 = jnp.float32
    outs = (jax.ShapeDtypeStruct((_NPAD, 128), f32),
            jax.ShapeDtypeStruct((_NPAD, 128), f32),
            jax.ShapeDtypeStruct((2 * _TILES, _RPT), f32))
    scratch = [
        pltpu.VMEM((_G0, 2, _B), jnp.int32),
        pltpu.VMEM((_G0, 2, _B), jnp.int32),
        pltpu.VMEM((_B, 128), f32),
        pltpu.VMEM((_B, 128), f32),
        pltpu.SemaphoreType.DMA,
        pltpu.SemaphoreType.DMA,
        pltpu.SemaphoreType.DMA,
        pltpu.SemaphoreType.DMA,
        pltpu.SemaphoreType.DMA,
        pltpu.SemaphoreType.DMA,
        pltpu.SemaphoreType.DMA,
        pltpu.VMEM_SHARED((_NPAD, 128), f32),
        pltpu.VMEM_SHARED((_NPAD,), f32),
        pltpu.VMEM((_B,), f32),
        pltpu.VMEM((_RPT,), f32),
    ]
    mesh = plsc.VectorSubcoreMesh(core_axis_name="c", subcore_axis_name="s")

    def body(z2, z1, edges, xf, p0, p1, degp,
             ib0, ib1, ra, rb, is0, is1, ga, gb, sa, sb, dsem,
             acc, deg_acc, ones_v, dz_v):
        c = lax.axis_index("c")
        t = lax.axis_index("s")
        wid = c * _TILES + t
        r0 = t * _RPT
        pltpu.sync_copy(z2.at[pl.ds(r0, _RPT)], acc.at[pl.ds(r0, _RPT)])
        pltpu.sync_copy(z1.at[pl.ds(r0, _RPT)], dz_v)
        pltpu.sync_copy(dz_v, deg_acc.at[pl.ds(r0, _RPT)])
        for k in range(_B // 16):
            ones_v[pl.ds(k * 16, 16)] = jnp.ones((16,), f32)
        plsc.subcore_barrier()
        _run_groups(edges.at[wid], xf, acc, (ib0, ib1), (is0, is1),
                    (ra, rb), (ga, gb), (sa, sb), _G0, _CH0 // _G0,
                    deg=(deg_acc, ones_v, dsem))
        plsc.subcore_barrier()
        for cc in range(2):
            out = (p0, p1)[cc]

            @pl.when(c == cc)
            def _():
                pltpu.sync_copy(acc.at[pl.ds(r0, _RPT)],
                                out.at[pl.ds(r0, _RPT)])
        pltpu.sync_copy(deg_acc.at[pl.ds(r0, _RPT)], dz_v)
        pltpu.sync_copy(dz_v, degp.at[wid])

    return pl.kernel(body, out_type=outs, mesh=mesh, scratch_types=scratch)


def _make_sc_agg():
    f32 = jnp.float32
    outs = (jax.ShapeDtypeStruct((_NPAD, 128), f32),
            jax.ShapeDtypeStruct((_NPAD, 128), f32))
    scratch = [
        pltpu.VMEM((_G, 2, _B), jnp.int32),
        pltpu.VMEM((_G, 2, _B), jnp.int32),
        pltpu.VMEM((_B, 128), f32),
        pltpu.VMEM((_B, 128), f32),
        pltpu.SemaphoreType.DMA,
        pltpu.SemaphoreType.DMA,
        pltpu.SemaphoreType.DMA,
        pltpu.SemaphoreType.DMA,
        pltpu.SemaphoreType.DMA,
        pltpu.SemaphoreType.DMA,
        pltpu.VMEM_SHARED((_NPAD, 128), f32),
    ]
    mesh = plsc.VectorSubcoreMesh(core_axis_name="c", subcore_axis_name="s")

    def body(z2, edges, t0, t1, a0, a1,
             ib0, ib1, ra, rb, is0, is1, ga, gb, sa, sb, acc):
        c = lax.axis_index("c")
        t = lax.axis_index("s")
        r0 = t * _RPT
        pltpu.sync_copy(z2.at[pl.ds(r0, _RPT)], acc.at[pl.ds(r0, _RPT)])
        plsc.subcore_barrier()
        for cc in range(2):
            tbl = (t0, t1)[cc]
            out = (a0, a1)[cc]

            @pl.when(c == cc)
            def _():
                _run_groups(edges.at[t], tbl, acc, (ib0, ib1), (is0, is1),
                            (ra, rb), (ga, gb), (sa, sb), _G, _CH // _G)
                plsc.subcore_barrier()
                pltpu.sync_copy(acc.at[pl.ds(r0, _RPT)],
                                out.at[pl.ds(r0, _RPT)])

    return pl.kernel(body, out_type=outs, mesh=mesh, scratch_types=scratch)


_BN = 1000
_full = lambda shape: pl.BlockSpec(shape, lambda i: (0, 0))
_rows = lambda w: pl.BlockSpec((_BN, w), lambda i: (i, 0))


def _make_tc_layer0():
    f32 = jnp.float32
    hh = _H // 2

    def body(p0, p1, xf, d0, d1, wl, wr, b, o0, o1, sc_o):
        scale = 1.0 / jnp.maximum(d0[...] + d1[...], 1.0)
        acc = (jnp.dot((p0[...] + p1[...]) * scale, wl[...],
                       preferred_element_type=f32)
               + jnp.dot(xf[...], wr[...], preferred_element_type=f32)
               + b[...])
        acc = jnp.maximum(acc, 0.0)
        o0[...] = acc[:, :hh]
        o1[...] = acc[:, hh:]
        sc_o[...] = scale

    return pl.pallas_call(
        body,
        grid=(_N // _BN,),
        in_specs=[_rows(128), _rows(128), _rows(128), _rows(1), _rows(1),
                  _full((128, _H)), _full((128, _H)), _full((1, _H))],
        out_specs=[_rows(hh), _rows(hh), _rows(1)],
        out_shape=[jax.ShapeDtypeStruct((_N, hh), f32),
                   jax.ShapeDtypeStruct((_N, hh), f32),
                   jax.ShapeDtypeStruct((_N, 1), f32)],
    )


def _make_tc_layer(relu):
    f32 = jnp.float32
    dh = 128
    hh = _H // 2

    def body(a0, a1, h0, h1, sc, wl, wr, b, o0, o1):
        scale = sc[...]
        acc = (jnp.dot(a0[...] * scale, wl[0:dh, :], preferred_element_type=f32)
               + jnp.dot(a1[...] * scale, wl[dh:2 * dh, :], preferred_element_type=f32)
               + jnp.dot(h0[...], wr[0:dh, :], preferred_element_type=f32)
               + jnp.dot(h1[...], wr[dh:2 * dh, :], preferred_element_type=f32)
               + b[...])
        if relu:
            acc = jnp.maximum(acc, 0.0)
        o0[...] = acc[:, :hh]
        o1[...] = acc[:, hh:]

    return pl.pallas_call(
        body,
        grid=(_N // _BN,),
        in_specs=[_rows(dh), _rows(dh), _rows(dh), _rows(dh), _rows(1),
                  _full((2 * dh, _H)), _full((2 * dh, _H)), _full((1, _H))],
        out_specs=[_rows(hh), _rows(hh)],
        out_shape=[jax.ShapeDtypeStruct((_N, hh), f32),
                   jax.ShapeDtypeStruct((_N, hh), f32)],
    )


def _make_tc_final():
    f32 = jnp.float32
    dh = 128

    def body(a0, a1, h0, h1, sc, wl, wr, b,
             ws1, bs1, ws2, bs2, wn1, bn1, wn2, bn2, seg_o, nud_o):
        scale = sc[...]
        h = (jnp.dot(a0[...] * scale, wl[0:dh, :], preferred_element_type=f32)
             + jnp.dot(a1[...] * scale, wl[dh:2 * dh, :], preferred_element_type=f32)
             + jnp.dot(h0[...], wr[0:dh, :], preferred_element_type=f32)
             + jnp.dot(h1[...], wr[dh:2 * dh, :], preferred_element_type=f32)
             + b[...])
        sh = jnp.maximum(jnp.dot(h, ws1[...], preferred_element_type=f32)
                         + bs1[...], 0.0)
        seg_o[...] = jnp.dot(sh, ws2[...], preferred_element_type=f32) + bs2[...]
        nh = jnp.maximum(jnp.dot(h, wn1[...], preferred_element_type=f32)
                         + bn1[...], 0.0)
        nud_o[...] = jnp.dot(nh, wn2[...], preferred_element_type=f32) + bn2[...]

    return pl.pallas_call(
        body,
        grid=(_N // _BN,),
        in_specs=[_rows(dh), _rows(dh), _rows(dh), _rows(dh), _rows(1),
                  _full((2 * dh, _H)), _full((2 * dh, _H)), _full((1, _H)),
                  _full((_H, _H)), _full((1, _H)), _full((_H, 6)), _full((1, 6)),
                  _full((_H, _H)), _full((1, _H)), _full((_H, 5)), _full((1, 5))],
        out_specs=[_rows(6), _rows(5)],
        out_shape=[jax.ShapeDtypeStruct((_N, 6), f32),
                   jax.ShapeDtypeStruct((_N, 5), f32)],
    )


def kernel(x, edge_index, W0_l, W0_r, b0, W1_l, W1_r, b1, W2_l, W2_r, b2,
           Ws1, bs1, Ws2, bs2, Wn1, bn1, Wn2, bn2):
    f32 = jnp.float32
    src = edge_index[0]
    dst = edge_index[1]
    srcs0 = jnp.pad(src.reshape(2 * _TILES, _EPT0),
                    ((0, 0), (0, _CH0 * _B - _EPT0))).reshape(
                        2 * _TILES, _CH0, 1, _B)
    dsts0 = jnp.pad(dst.reshape(2 * _TILES, _EPT0),
                    ((0, 0), (0, _CH0 * _B - _EPT0)),
                    constant_values=_N).reshape(2 * _TILES, _CH0, 1, _B)
    edges0 = jnp.concatenate([srcs0, dsts0], axis=2)
    srcs = jnp.pad(src.reshape(_TILES, _EPT),
                   ((0, 0), (0, _CH * _B - _EPT))).reshape(_TILES, _CH, 1, _B)
    dsts = jnp.pad(dst.reshape(_TILES, _EPT),
                   ((0, 0), (0, _CH * _B - _EPT)),
                   constant_values=_N).reshape(_TILES, _CH, 1, _B)
    edges = jnp.concatenate([srcs, dsts], axis=2)
    z128 = jnp.zeros((_NPAD, 128), f32)
    z1d = jnp.zeros((_NPAD,), f32)

    p0, p1, degp = _make_sc_agg0()(z128, z1d, edges0, x)
    d0 = degp[:_TILES].reshape(-1)[:_N].reshape(_N, 1)
    d1 = degp[_TILES:].reshape(-1)[:_N].reshape(_N, 1)

    h0, h1, scale = _make_tc_layer0()(
        p0[:_N], p1[:_N], x, d0, d1, W0_l, W0_r, b0.reshape(1, -1))

    sc_mid = _make_sc_agg()
    a0p, a1p = sc_mid(z128, edges, h0, h1)
    h0, h1 = _make_tc_layer(True)(
        a0p[:_N], a1p[:_N], h0, h1, scale, W1_l, W1_r, b1.reshape(1, -1))

    a0p, a1p = sc_mid(z128, edges, h0, h1)
    seg, nud = _make_tc_final()(
        a0p[:_N], a1p[:_N], h0, h1, scale, W2_l, W2_r, b2.reshape(1, -1),
        Ws1, bs1.reshape(1, -1), Ws2, bs2.reshape(1, -1),
        Wn1, bn1.reshape(1, -1), Wn2, bn2.reshape(1, -1))
    return (seg, nud)


# speedup vs baseline: 4.0009x; 1.0127x over previous
import jax
import jax.numpy as jnp
from jax import lax
from jax.experimental import pallas as pl
from jax.experimental.pallas import tpu as pltpu
from jax.experimental.pallas import tpu_sc as plsc

_N = 10000
_H = 256
_TILES = 16
_B = 128
_EPT = 20000
_CH = 160
_G = 16
_EPT0 = 10000
_CH0 = 80
_G0 = 8
_NPAD = 10112
_RPT = _NPAD // _TILES


def _emit_group(tbl, acc, ibuf, bufs, gsems, ssems, ng, deg=None):
    gdesc = [None, None]
    sdesc = [None, None]
    ddesc = []
    gdesc[0] = pltpu.async_copy(tbl.at[ibuf.at[0, 0]], bufs[0], gsems[0])
    for k in range(ng):
        p = k % 2
        q = (k + 1) % 2
        if k + 1 < ng:
            if sdesc[q] is not None:
                sdesc[q].wait()
            gdesc[q] = pltpu.async_copy(tbl.at[ibuf.at[k + 1, 0]],
                                        bufs[q], gsems[q])
        gdesc[p].wait()
        sdesc[p] = pltpu.async_copy(bufs[p], acc.at[ibuf.at[k, 1]],
                                    ssems[p], add=True)
        if deg is not None:
            deg_acc, ones_v, dsem = deg
            ddesc.append(pltpu.async_copy(ones_v, deg_acc.at[ibuf.at[k, 1]],
                                          dsem, add=True))
    sdesc[0].wait()
    sdesc[1].wait()
    for d in ddesc:
        d.wait()


def _run_groups(edges_t, tbl, acc, ibufs, isems, bufs, gsems, ssems,
                ng, n_groups, idx0=None, deg=None):
    if idx0 is None:
        pltpu.sync_copy(edges_t.at[pl.ds(0, ng)], ibufs[0])
    else:
        idx0.wait()

    def pair(m, carry):
        for par in range(2):
            g = 2 * m + par
            cur = ibufs[par]
            nxt = ibufs[1 - par]

            @pl.when(g > 0)
            def _():
                pltpu.make_async_copy(edges_t.at[pl.ds(0, ng)], cur,
                                      isems[par]).wait()

            @pl.when(g + 1 < n_groups)
            def _():
                pltpu.async_copy(edges_t.at[pl.ds((g + 1) * ng, ng)], nxt,
                                 isems[1 - par])
            _emit_group(tbl, acc, cur, bufs, gsems, ssems, ng, deg)
        return carry
    lax.fori_loop(0, n_groups // 2, pair, 0)


def _make_sc_agg0():
    f32 = jnp.float32
    outs = (jax.ShapeDtypeStruct((_NPAD, 128), f32),
            jax.ShapeDtypeStruct((_NPAD, 128), f32),
            jax.ShapeDtypeStruct((2 * _TILES, _RPT), f32))
    scratch = [
        pltpu.VMEM((_G0, 2, _B), jnp.int32),
        pltpu.VMEM((_G0, 2, _B), jnp.int32),
        pltpu.VMEM((_B, 128), f32),
        pltpu.VMEM((_B, 128), f32),
        pltpu.SemaphoreType.DMA,
        pltpu.SemaphoreType.DMA,
        pltpu.SemaphoreType.DMA,
        pltpu.SemaphoreType.DMA,
        pltpu.SemaphoreType.DMA,
        pltpu.SemaphoreType.DMA,
        pltpu.SemaphoreType.DMA,
        pltpu.VMEM_SHARED((_NPAD, 128), f32),
        pltpu.VMEM_SHARED((_NPAD,), f32),
        pltpu.VMEM((_B,), f32),
        pltpu.VMEM((_RPT,), f32),
    ]
    mesh = plsc.VectorSubcoreMesh(core_axis_name="c", subcore_axis_name="s")

    def body(z2, z1, edges, xf, p0, p1, degp,
             ib0, ib1, ra, rb, is0, is1, ga, gb, sa, sb, dsem,
             acc, deg_acc, ones_v, dz_v):
        c = lax.axis_index("c")
        t = lax.axis_index("s")
        wid = c * _TILES + t
        r0 = t * _RPT
        idx0 = pltpu.async_copy(edges.at[wid].at[pl.ds(0, _G0)], ib0, is0)
        pltpu.sync_copy(z2.at[pl.ds(r0, _RPT)], acc.at[pl.ds(r0, _RPT)])
        pltpu.sync_copy(z1.at[pl.ds(r0, _RPT)], dz_v)
        pltpu.sync_copy(dz_v, deg_acc.at[pl.ds(r0, _RPT)])
        for k in range(_B // 16):
            ones_v[pl.ds(k * 16, 16)] = jnp.ones((16,), f32)
        plsc.subcore_barrier()
        _run_groups(edges.at[wid], xf, acc, (ib0, ib1), (is0, is1),
                    (ra, rb), (ga, gb), (sa, sb), _G0, _CH0 // _G0,
                    idx0=idx0, deg=(deg_acc, ones_v, dsem))
        plsc.subcore_barrier()
        for cc in range(2):
            out = (p0, p1)[cc]

            @pl.when(c == cc)
            def _():
                pltpu.sync_copy(acc.at[pl.ds(r0, _RPT)],
                                out.at[pl.ds(r0, _RPT)])
        pltpu.sync_copy(deg_acc.at[pl.ds(r0, _RPT)], dz_v)
        pltpu.sync_copy(dz_v, degp.at[wid])

    return pl.kernel(body, out_type=outs, mesh=mesh, scratch_types=scratch)


def _make_sc_agg():
    f32 = jnp.float32
    outs = (jax.ShapeDtypeStruct((_NPAD, 128), f32),
            jax.ShapeDtypeStruct((_NPAD, 128), f32))
    scratch = [
        pltpu.VMEM((_G, 2, _B), jnp.int32),
        pltpu.VMEM((_G, 2, _B), jnp.int32),
        pltpu.VMEM((_B, 128), f32),
        pltpu.VMEM((_B, 128), f32),
        pltpu.SemaphoreType.DMA,
        pltpu.SemaphoreType.DMA,
        pltpu.SemaphoreType.DMA,
        pltpu.SemaphoreType.DMA,
        pltpu.SemaphoreType.DMA,
        pltpu.SemaphoreType.DMA,
        pltpu.VMEM_SHARED((_NPAD, 128), f32),
    ]
    mesh = plsc.VectorSubcoreMesh(core_axis_name="c", subcore_axis_name="s")

    def body(z2, edges, t0, t1, a0, a1,
             ib0, ib1, ra, rb, is0, is1, ga, gb, sa, sb, acc):
        c = lax.axis_index("c")
        t = lax.axis_index("s")
        r0 = t * _RPT
        idx0 = pltpu.async_copy(edges.at[t].at[pl.ds(0, _G)], ib0, is0)
        pltpu.sync_copy(z2.at[pl.ds(r0, _RPT)], acc.at[pl.ds(r0, _RPT)])
        plsc.subcore_barrier()
        for cc in range(2):
            tbl = (t0, t1)[cc]
            out = (a0, a1)[cc]

            @pl.when(c == cc)
            def _():
                _run_groups(edges.at[t], tbl, acc, (ib0, ib1), (is0, is1),
                            (ra, rb), (ga, gb), (sa, sb), _G, _CH // _G,
                            idx0=idx0)
                plsc.subcore_barrier()
                pltpu.sync_copy(acc.at[pl.ds(r0, _RPT)],
                                out.at[pl.ds(r0, _RPT)])

    return pl.kernel(body, out_type=outs, mesh=mesh, scratch_types=scratch)


_BN = 1000
_full = lambda shape: pl.BlockSpec(shape, lambda i: (0, 0))
_rows = lambda w: pl.BlockSpec((_BN, w), lambda i: (i, 0))


def _make_tc_layer0():
    f32 = jnp.float32
    hh = _H // 2

    def body(p0, p1, xf, d0, d1, wl, wr, b, o0, o1, sc_o):
        scale = 1.0 / jnp.maximum(d0[...] + d1[...], 1.0)
        acc = (jnp.dot((p0[...] + p1[...]) * scale, wl[...],
                       preferred_element_type=f32)
               + jnp.dot(xf[...], wr[...], preferred_element_type=f32)
               + b[...])
        acc = jnp.maximum(acc, 0.0)
        o0[...] = acc[:, :hh]
        o1[...] = acc[:, hh:]
        sc_o[...] = scale

    return pl.pallas_call(
        body,
        grid=(_N // _BN,),
        in_specs=[_rows(128), _rows(128), _rows(128), _rows(1), _rows(1),
                  _full((128, _H)), _full((128, _H)), _full((1, _H))],
        out_specs=[_rows(hh), _rows(hh), _rows(1)],
        out_shape=[jax.ShapeDtypeStruct((_N, hh), f32),
                   jax.ShapeDtypeStruct((_N, hh), f32),
                   jax.ShapeDtypeStruct((_N, 1), f32)],
    )


def _make_tc_layer(relu):
    f32 = jnp.float32
    dh = 128
    hh = _H // 2

    def body(a0, a1, h0, h1, sc, wl, wr, b, o0, o1):
        scale = sc[...]
        acc = (jnp.dot(a0[...] * scale, wl[0:dh, :], preferred_element_type=f32)
               + jnp.dot(a1[...] * scale, wl[dh:2 * dh, :], preferred_element_type=f32)
               + jnp.dot(h0[...], wr[0:dh, :], preferred_element_type=f32)
               + jnp.dot(h1[...], wr[dh:2 * dh, :], preferred_element_type=f32)
               + b[...])
        if relu:
            acc = jnp.maximum(acc, 0.0)
        o0[...] = acc[:, :hh]
        o1[...] = acc[:, hh:]

    return pl.pallas_call(
        body,
        grid=(_N // _BN,),
        in_specs=[_rows(dh), _rows(dh), _rows(dh), _rows(dh), _rows(1),
                  _full((2 * dh, _H)), _full((2 * dh, _H)), _full((1, _H))],
        out_specs=[_rows(hh), _rows(hh)],
        out_shape=[jax.ShapeDtypeStruct((_N, hh), f32),
                   jax.ShapeDtypeStruct((_N, hh), f32)],
    )


def _make_tc_final():
    f32 = jnp.float32
    dh = 128

    def body(a0, a1, h0, h1, sc, wl, wr, b,
             ws1, bs1, ws2, bs2, wn1, bn1, wn2, bn2, seg_o, nud_o):
        scale = sc[...]
        h = (jnp.dot(a0[...] * scale, wl[0:dh, :], preferred_element_type=f32)
             + jnp.dot(a1[...] * scale, wl[dh:2 * dh, :], preferred_element_type=f32)
             + jnp.dot(h0[...], wr[0:dh, :], preferred_element_type=f32)
             + jnp.dot(h1[...], wr[dh:2 * dh, :], preferred_element_type=f32)
             + b[...])
        sh = jnp.maximum(jnp.dot(h, ws1[...], preferred_element_type=f32)
                         + bs1[...], 0.0)
        seg_o[...] = jnp.dot(sh, ws2[...], preferred_element_type=f32) + bs2[...]
        nh = jnp.maximum(jnp.dot(h, wn1[...], preferred_element_type=f32)
                         + bn1[...], 0.0)
        nud_o[...] = jnp.dot(nh, wn2[...], preferred_element_type=f32) + bn2[...]

    return pl.pallas_call(
        body,
        grid=(_N // _BN,),
        in_specs=[_rows(dh), _rows(dh), _rows(dh), _rows(dh), _rows(1),
                  _full((2 * dh, _H)), _full((2 * dh, _H)), _full((1, _H)),
                  _full((_H, _H)), _full((1, _H)), _full((_H, 6)), _full((1, 6)),
                  _full((_H, _H)), _full((1, _H)), _full((_H, 5)), _full((1, 5))],
        out_specs=[_rows(6), _rows(5)],
        out_shape=[jax.ShapeDtypeStruct((_N, 6), f32),
                   jax.ShapeDtypeStruct((_N, 5), f32)],
    )


def kernel(x, edge_index, W0_l, W0_r, b0, W1_l, W1_r, b1, W2_l, W2_r, b2,
           Ws1, bs1, Ws2, bs2, Wn1, bn1, Wn2, bn2):
    f32 = jnp.float32
    src = edge_index[0]
    dst = edge_index[1]
    srcs0 = jnp.pad(src.reshape(2 * _TILES, _EPT0),
                    ((0, 0), (0, _CH0 * _B - _EPT0))).reshape(
                        2 * _TILES, _CH0, 1, _B)
    dsts0 = jnp.pad(dst.reshape(2 * _TILES, _EPT0),
                    ((0, 0), (0, _CH0 * _B - _EPT0)),
                    constant_values=_N).reshape(2 * _TILES, _CH0, 1, _B)
    edges0 = jnp.concatenate([srcs0, dsts0], axis=2)
    srcs = jnp.pad(src.reshape(_TILES, _EPT),
                   ((0, 0), (0, _CH * _B - _EPT))).reshape(_TILES, _CH, 1, _B)
    dsts = jnp.pad(dst.reshape(_TILES, _EPT),
                   ((0, 0), (0, _CH * _B - _EPT)),
                   constant_values=_N).reshape(_TILES, _CH, 1, _B)
    edges = jnp.concatenate([srcs, dsts], axis=2)
    z128 = jnp.zeros((_NPAD, 128), f32)
    z1d = jnp.zeros((_NPAD,), f32)

    p0, p1, degp = _make_sc_agg0()(z128, z1d, edges0, x)
    d0 = degp[:_TILES].reshape(-1)[:_N].reshape(_N, 1)
    d1 = degp[_TILES:].reshape(-1)[:_N].reshape(_N, 1)

    h0, h1, scale = _make_tc_layer0()(
        p0, p1, x, d0, d1, W0_l, W0_r, b0.reshape(1, -1))

    sc_mid = _make_sc_agg()
    a0p, a1p = sc_mid(z128, edges, h0, h1)
    h0, h1 = _make_tc_layer(True)(
        a0p, a1p, h0, h1, scale, W1_l, W1_r, b1.reshape(1, -1))

    a0p, a1p = sc_mid(z128, edges, h0, h1)
    seg, nud = _make_tc_final()(
        a0p, a1p, h0, h1, scale, W2_l, W2_r, b2.reshape(1, -1),
        Ws1, bs1.reshape(1, -1), Ws2, bs2.reshape(1, -1),
        Wn1, bn1.reshape(1, -1), Wn2, bn2.reshape(1, -1))
    return (seg, nud)


# speedup vs baseline: 4.0057x; 1.0012x over previous
import jax
import jax.numpy as jnp
from jax import lax
from jax.experimental import pallas as pl
from jax.experimental.pallas import tpu as pltpu
from jax.experimental.pallas import tpu_sc as plsc

_N = 10000
_H = 256
_TILES = 16
_B = 128
_EPT = 20000
_CH = 160
_G = 16
_EPT0 = 10000
_CH0 = 80
_G0 = 8
_NPAD = 10112
_RPT = _NPAD // _TILES


def _emit_group(tbl, acc, ibuf, bufs, gsems, ssems, ng, deg=None):
    gdesc = [None, None]
    sdesc = [None, None]
    ddesc = []
    gdesc[0] = pltpu.async_copy(tbl.at[ibuf.at[0, 0]], bufs[0], gsems[0])
    for k in range(ng):
        p = k % 2
        q = (k + 1) % 2
        if k + 1 < ng:
            if sdesc[q] is not None:
                sdesc[q].wait()
            gdesc[q] = pltpu.async_copy(tbl.at[ibuf.at[k + 1, 0]],
                                        bufs[q], gsems[q])
        gdesc[p].wait()
        sdesc[p] = pltpu.async_copy(bufs[p], acc.at[ibuf.at[k, 1]],
                                    ssems[p], add=True)
        if deg is not None:
            deg_acc, ones_v, dsem = deg
            ddesc.append(pltpu.async_copy(ones_v, deg_acc.at[ibuf.at[k, 1]],
                                          dsem, add=True))
    sdesc[0].wait()
    sdesc[1].wait()
    for d in ddesc:
        d.wait()


def _run_groups(edges_t, tbl, acc, ibufs, isems, bufs, gsems, ssems,
                ng, n_groups, idx0=None, deg=None):
    if idx0 is None:
        pltpu.sync_copy(edges_t.at[pl.ds(0, ng)], ibufs[0])
    else:
        idx0.wait()

    def pair(m, carry):
        for par in range(2):
            g = 2 * m + par
            cur = ibufs[par]
            nxt = ibufs[1 - par]

            @pl.when(g > 0)
            def _():
                pltpu.make_async_copy(edges_t.at[pl.ds(0, ng)], cur,
                                      isems[par]).wait()

            @pl.when(g + 1 < n_groups)
            def _():
                pltpu.async_copy(edges_t.at[pl.ds((g + 1) * ng, ng)], nxt,
                                 isems[1 - par])
            _emit_group(tbl, acc, cur, bufs, gsems, ssems, ng, deg)
        return carry
    lax.fori_loop(0, n_groups // 2, pair, 0)


def _make_sc_agg0():
    f32 = jnp.float32
    outs = (jax.ShapeDtypeStruct((_NPAD, 128), f32),
            jax.ShapeDtypeStruct((_NPAD, 128), f32),
            jax.ShapeDtypeStruct((2 * _TILES, _RPT), f32))
    scratch = [
        pltpu.VMEM((_G0, 2, _B), jnp.int32),
        pltpu.VMEM((_G0, 2, _B), jnp.int32),
        pltpu.VMEM((_B, 128), f32),
        pltpu.VMEM((_B, 128), f32),
        pltpu.SemaphoreType.DMA,
        pltpu.SemaphoreType.DMA,
        pltpu.SemaphoreType.DMA,
        pltpu.SemaphoreType.DMA,
        pltpu.SemaphoreType.DMA,
        pltpu.SemaphoreType.DMA,
        pltpu.SemaphoreType.DMA,
        pltpu.VMEM_SHARED((_NPAD, 128), f32),
        pltpu.VMEM_SHARED((_NPAD,), f32),
        pltpu.VMEM((_B,), f32),
        pltpu.VMEM((_RPT,), f32),
    ]
    mesh = plsc.VectorSubcoreMesh(core_axis_name="c", subcore_axis_name="s")

    def body(z2, z1, edges, xf, p0, p1, degp,
             ib0, ib1, ra, rb, is0, is1, ga, gb, sa, sb, dsem,
             acc, deg_acc, ones_v, dz_v):
        c = lax.axis_index("c")
        t = lax.axis_index("s")
        wid = c * _TILES + t
        r0 = t * _RPT
        idx0 = pltpu.async_copy(edges.at[wid].at[pl.ds(0, _G0)], ib0, is0)
        pltpu.sync_copy(z2.at[pl.ds(r0, _RPT)], acc.at[pl.ds(r0, _RPT)])
        pltpu.sync_copy(z1.at[pl.ds(r0, _RPT)], dz_v)
        pltpu.sync_copy(dz_v, deg_acc.at[pl.ds(r0, _RPT)])
        for k in range(_B // 16):
            ones_v[pl.ds(k * 16, 16)] = jnp.ones((16,), f32)
        plsc.subcore_barrier()
        _run_groups(edges.at[wid], xf, acc, (ib0, ib1), (is0, is1),
                    (ra, rb), (ga, gb), (sa, sb), _G0, _CH0 // _G0,
                    idx0=idx0, deg=(deg_acc, ones_v, dsem))
        plsc.subcore_barrier()
        for cc in range(2):
            out = (p0, p1)[cc]

            @pl.when(c == cc)
            def _():
                pltpu.sync_copy(acc.at[pl.ds(r0, _RPT)],
                                out.at[pl.ds(r0, _RPT)])
        pltpu.sync_copy(deg_acc.at[pl.ds(r0, _RPT)], dz_v)
        pltpu.sync_copy(dz_v, degp.at[wid])

    return pl.kernel(body, out_type=outs, mesh=mesh, scratch_types=scratch)


def _make_sc_agg():
    f32 = jnp.float32
    outs = (jax.ShapeDtypeStruct((_NPAD, 128), f32),
            jax.ShapeDtypeStruct((_NPAD, 128), f32))
    scratch = [
        pltpu.VMEM((_G, 2, _B), jnp.int32),
        pltpu.VMEM((_G, 2, _B), jnp.int32),
        pltpu.VMEM((_B, 128), f32),
        pltpu.VMEM((_B, 128), f32),
        pltpu.SemaphoreType.DMA,
        pltpu.SemaphoreType.DMA,
        pltpu.SemaphoreType.DMA,
        pltpu.SemaphoreType.DMA,
        pltpu.SemaphoreType.DMA,
        pltpu.SemaphoreType.DMA,
        pltpu.VMEM_SHARED((_NPAD, 128), f32),
    ]
    mesh = plsc.VectorSubcoreMesh(core_axis_name="c", subcore_axis_name="s")

    def body(z2, edges, t0, t1, a0, a1,
             ib0, ib1, ra, rb, is0, is1, ga, gb, sa, sb, acc):
        c = lax.axis_index("c")
        t = lax.axis_index("s")
        r0 = t * _RPT
        idx0 = pltpu.async_copy(edges.at[t].at[pl.ds(0, _G)], ib0, is0)
        pltpu.sync_copy(z2.at[pl.ds(r0, _RPT)], acc.at[pl.ds(r0, _RPT)])
        plsc.subcore_barrier()
        for cc in range(2):
            tbl = (t0, t1)[cc]
            out = (a0, a1)[cc]

            @pl.when(c == cc)
            def _():
                _run_groups(edges.at[t], tbl, acc, (ib0, ib1), (is0, is1),
                            (ra, rb), (ga, gb), (sa, sb), _G, _CH // _G,
                            idx0=idx0)
                plsc.subcore_barrier()
                pltpu.sync_copy(acc.at[pl.ds(r0, _RPT)],
                                out.at[pl.ds(r0, _RPT)])

    return pl.kernel(body, out_type=outs, mesh=mesh, scratch_types=scratch)


_BN = 1000
_full = lambda shape: pl.BlockSpec(shape, lambda i: (0, 0))
_rows = lambda w: pl.BlockSpec((_BN, w), lambda i: (i, 0))


def _make_tc_r0():
    f32 = jnp.float32

    def body(xf, wr, b, r_o):
        r_o[...] = jnp.dot(xf[...], wr[...], preferred_element_type=f32) + b[...]

    return pl.pallas_call(
        body,
        grid=(_N // _BN,),
        in_specs=[_rows(128), _full((128, _H)), _full((1, _H))],
        out_specs=[_rows(_H)],
        out_shape=[jax.ShapeDtypeStruct((_N, _H), f32)],
    )


def _make_tc_r():
    f32 = jnp.float32
    dh = 128

    def body(h0, h1, wr, b, r_o):
        r_o[...] = (jnp.dot(h0[...], wr[0:dh, :], preferred_element_type=f32)
                    + jnp.dot(h1[...], wr[dh:2 * dh, :],
                              preferred_element_type=f32) + b[...])

    return pl.pallas_call(
        body,
        grid=(_N // _BN,),
        in_specs=[_rows(dh), _rows(dh), _full((2 * dh, _H)), _full((1, _H))],
        out_specs=[_rows(_H)],
        out_shape=[jax.ShapeDtypeStruct((_N, _H), f32)],
    )


def _make_tc_layer0():
    f32 = jnp.float32
    hh = _H // 2

    def body(p0, p1, r, d0, d1, wl, o0, o1, sc_o):
        scale = 1.0 / jnp.maximum(d0[...] + d1[...], 1.0)
        acc = jnp.dot((p0[...] + p1[...]) * scale, wl[...],
                      preferred_element_type=f32) + r[...]
        acc = jnp.maximum(acc, 0.0)
        o0[...] = acc[:, :hh]
        o1[...] = acc[:, hh:]
        sc_o[...] = scale

    return pl.pallas_call(
        body,
        grid=(_N // _BN,),
        in_specs=[_rows(128), _rows(128), _rows(_H), _rows(1), _rows(1),
                  _full((128, _H))],
        out_specs=[_rows(hh), _rows(hh), _rows(1)],
        out_shape=[jax.ShapeDtypeStruct((_N, hh), f32),
                   jax.ShapeDtypeStruct((_N, hh), f32),
                   jax.ShapeDtypeStruct((_N, 1), f32)],
    )


def _make_tc_layer(relu):
    f32 = jnp.float32
    dh = 128
    hh = _H // 2

    def body(a0, a1, r, sc, wl, o0, o1):
        scale = sc[...]
        acc = (jnp.dot(a0[...] * scale, wl[0:dh, :], preferred_element_type=f32)
               + jnp.dot(a1[...] * scale, wl[dh:2 * dh, :],
                         preferred_element_type=f32)
               + r[...])
        if relu:
            acc = jnp.maximum(acc, 0.0)
        o0[...] = acc[:, :hh]
        o1[...] = acc[:, hh:]

    return pl.pallas_call(
        body,
        grid=(_N // _BN,),
        in_specs=[_rows(dh), _rows(dh), _rows(_H), _rows(1),
                  _full((2 * dh, _H))],
        out_specs=[_rows(hh), _rows(hh)],
        out_shape=[jax.ShapeDtypeStruct((_N, hh), f32),
                   jax.ShapeDtypeStruct((_N, hh), f32)],
    )


def _make_tc_final():
    f32 = jnp.float32
    dh = 128

    def body(a0, a1, r, sc, wl,
             ws1, bs1, ws2, bs2, wn1, bn1, wn2, bn2, seg_o, nud_o):
        scale = sc[...]
        h = (jnp.dot(a0[...] * scale, wl[0:dh, :], preferred_element_type=f32)
             + jnp.dot(a1[...] * scale, wl[dh:2 * dh, :],
                       preferred_element_type=f32)
             + r[...])
        sh = jnp.maximum(jnp.dot(h, ws1[...], preferred_element_type=f32)
                         + bs1[...], 0.0)
        seg_o[...] = jnp.dot(sh, ws2[...], preferred_element_type=f32) + bs2[...]
        nh = jnp.maximum(jnp.dot(h, wn1[...], preferred_element_type=f32)
                         + bn1[...], 0.0)
        nud_o[...] = jnp.dot(nh, wn2[...], preferred_element_type=f32) + bn2[...]

    return pl.pallas_call(
        body,
        grid=(_N // _BN,),
        in_specs=[_rows(dh), _rows(dh), _rows(_H), _rows(1),
                  _full((2 * dh, _H)),
                  _full((_H, _H)), _full((1, _H)), _full((_H, 6)), _full((1, 6)),
                  _full((_H, _H)), _full((1, _H)), _full((_H, 5)), _full((1, 5))],
        out_specs=[_rows(6), _rows(5)],
        out_shape=[jax.ShapeDtypeStruct((_N, 6), f32),
                   jax.ShapeDtypeStruct((_N, 5), f32)],
    )


def kernel(x, edge_index, W0_l, W0_r, b0, W1_l, W1_r, b1, W2_l, W2_r, b2,
           Ws1, bs1, Ws2, bs2, Wn1, bn1, Wn2, bn2):
    f32 = jnp.float32
    src = edge_index[0]
    dst = edge_index[1]
    srcs0 = jnp.pad(src.reshape(2 * _TILES, _EPT0),
                    ((0, 0), (0, _CH0 * _B - _EPT0))).reshape(
                        2 * _TILES, _CH0, 1, _B)
    dsts0 = jnp.pad(dst.reshape(2 * _TILES, _EPT0),
                    ((0, 0), (0, _CH0 * _B - _EPT0)),
                    constant_values=_N).reshape(2 * _TILES, _CH0, 1, _B)
    edges0 = jnp.concatenate([srcs0, dsts0], axis=2)
    srcs = jnp.pad(src.reshape(_TILES, _EPT),
                   ((0, 0), (0, _CH * _B - _EPT))).reshape(_TILES, _CH, 1, _B)
    dsts = jnp.pad(dst.reshape(_TILES, _EPT),
                   ((0, 0), (0, _CH * _B - _EPT)),
                   constant_values=_N).reshape(_TILES, _CH, 1, _B)
    edges = jnp.concatenate([srcs, dsts], axis=2)
    z128 = jnp.zeros((_NPAD, 128), f32)
    z1d = jnp.zeros((_NPAD,), f32)

    p0, p1, degp = _make_sc_agg0()(z128, z1d, edges0, x)
    (r,) = _make_tc_r0()(x, W0_r, b0.reshape(1, -1))
    d0 = degp[:_TILES].reshape(-1)[:_N].reshape(_N, 1)
    d1 = degp[_TILES:].reshape(-1)[:_N].reshape(_N, 1)

    h0, h1, scale = _make_tc_layer0()(p0, p1, r, d0, d1, W0_l)

    sc_mid = _make_sc_agg()
    tc_r = _make_tc_r()
    a0p, a1p = sc_mid(z128, edges, h0, h1)
    (r,) = tc_r(h0, h1, W1_r, b1.reshape(1, -1))
    h0, h1 = _make_tc_layer(True)(a0p, a1p, r, scale, W1_l)

    a0p, a1p = sc_mid(z128, edges, h0, h1)
    (r,) = tc_r(h0, h1, W2_r, b2.reshape(1, -1))
    seg, nud = _make_tc_final()(
        a0p, a1p, r, scale, W2_l,
        Ws1, bs1.reshape(1, -1), Ws2, bs2.reshape(1, -1),
        Wn1, bn1.reshape(1, -1), Wn2, bn2.reshape(1, -1))
    return (seg, nud)


# speedup vs baseline: 4.0223x; 1.0041x over previous
import jax
import jax.numpy as jnp
from jax import lax
from jax.experimental import pallas as pl
from jax.experimental.pallas import tpu as pltpu
from jax.experimental.pallas import tpu_sc as plsc

_N = 10000
_H = 256
_TILES = 16
_B = 128
_EPT = 20000
_CH = 160
_G = 20
_EPT0 = 10000
_CH0 = 80
_G0 = 10
_NPAD = 10112
_RPT = _NPAD // _TILES


def _emit_group(tbl, acc, ibuf, bufs, gsems, ssems, ng, deg=None):
    gdesc = [None, None]
    sdesc = [None, None]
    ddesc = []
    gdesc[0] = pltpu.async_copy(tbl.at[ibuf.at[0, 0]], bufs[0], gsems[0])
    for k in range(ng):
        p = k % 2
        q = (k + 1) % 2
        if k + 1 < ng:
            if sdesc[q] is not None:
                sdesc[q].wait()
            gdesc[q] = pltpu.async_copy(tbl.at[ibuf.at[k + 1, 0]],
                                        bufs[q], gsems[q])
        gdesc[p].wait()
        sdesc[p] = pltpu.async_copy(bufs[p], acc.at[ibuf.at[k, 1]],
                                    ssems[p], add=True)
        if deg is not None:
            deg_acc, ones_v, dsem = deg
            ddesc.append(pltpu.async_copy(ones_v, deg_acc.at[ibuf.at[k, 1]],
                                          dsem, add=True))
    sdesc[0].wait()
    sdesc[1].wait()
    for d in ddesc:
        d.wait()


def _run_groups(edges_t, tbl, acc, ibufs, isems, bufs, gsems, ssems,
                ng, n_groups, idx0=None, deg=None):
    if idx0 is None:
        pltpu.sync_copy(edges_t.at[pl.ds(0, ng)], ibufs[0])
    else:
        idx0.wait()

    def pair(m, carry):
        for par in range(2):
            g = 2 * m + par
            cur = ibufs[par]
            nxt = ibufs[1 - par]

            @pl.when(g > 0)
            def _():
                pltpu.make_async_copy(edges_t.at[pl.ds(0, ng)], cur,
                                      isems[par]).wait()

            @pl.when(g + 1 < n_groups)
            def _():
                pltpu.async_copy(edges_t.at[pl.ds((g + 1) * ng, ng)], nxt,
                                 isems[1 - par])
            _emit_group(tbl, acc, cur, bufs, gsems, ssems, ng, deg)
        return carry
    lax.fori_loop(0, n_groups // 2, pair, 0)


def _make_sc_agg0():
    f32 = jnp.float32
    outs = (jax.ShapeDtypeStruct((_NPAD, 128), f32),
            jax.ShapeDtypeStruct((_NPAD, 128), f32),
            jax.ShapeDtypeStruct((2 * _TILES, _RPT), f32))
    scratch = [
        pltpu.VMEM((_G0, 2, _B), jnp.int32),
        pltpu.VMEM((_G0, 2, _B), jnp.int32),
        pltpu.VMEM((_B, 128), f32),
        pltpu.VMEM((_B, 128), f32),
        pltpu.SemaphoreType.DMA,
        pltpu.SemaphoreType.DMA,
        pltpu.SemaphoreType.DMA,
        pltpu.SemaphoreType.DMA,
        pltpu.SemaphoreType.DMA,
        pltpu.SemaphoreType.DMA,
        pltpu.SemaphoreType.DMA,
        pltpu.VMEM_SHARED((_NPAD, 128), f32),
        pltpu.VMEM_SHARED((_NPAD,), f32),
        pltpu.VMEM((_B,), f32),
        pltpu.VMEM((_RPT,), f32),
    ]
    mesh = plsc.VectorSubcoreMesh(core_axis_name="c", subcore_axis_name="s")

    def body(z2, z1, edges, xf, p0, p1, degp,
             ib0, ib1, ra, rb, is0, is1, ga, gb, sa, sb, dsem,
             acc, deg_acc, ones_v, dz_v):
        c = lax.axis_index("c")
        t = lax.axis_index("s")
        wid = c * _TILES + t
        r0 = t * _RPT
        idx0 = pltpu.async_copy(edges.at[wid].at[pl.ds(0, _G0)], ib0, is0)
        pltpu.sync_copy(z2.at[pl.ds(r0, _RPT)], acc.at[pl.ds(r0, _RPT)])
        pltpu.sync_copy(z1.at[pl.ds(r0, _RPT)], dz_v)
        pltpu.sync_copy(dz_v, deg_acc.at[pl.ds(r0, _RPT)])
        for k in range(_B // 16):
            ones_v[pl.ds(k * 16, 16)] = jnp.ones((16,), f32)
        plsc.subcore_barrier()
        _run_groups(edges.at[wid], xf, acc, (ib0, ib1), (is0, is1),
                    (ra, rb), (ga, gb), (sa, sb), _G0, _CH0 // _G0,
                    idx0=idx0, deg=(deg_acc, ones_v, dsem))
        plsc.subcore_barrier()
        for cc in range(2):
            out = (p0, p1)[cc]

            @pl.when(c == cc)
            def _():
                pltpu.sync_copy(acc.at[pl.ds(r0, _RPT)],
                                out.at[pl.ds(r0, _RPT)])
        pltpu.sync_copy(deg_acc.at[pl.ds(r0, _RPT)], dz_v)
        pltpu.sync_copy(dz_v, degp.at[wid])

    return pl.kernel(body, out_type=outs, mesh=mesh, scratch_types=scratch)


def _make_sc_agg():
    f32 = jnp.float32
    outs = (jax.ShapeDtypeStruct((_NPAD, 128), f32),
            jax.ShapeDtypeStruct((_NPAD, 128), f32))
    scratch = [
        pltpu.VMEM((_G, 2, _B), jnp.int32),
        pltpu.VMEM((_G, 2, _B), jnp.int32),
        pltpu.VMEM((_B, 128), f32),
        pltpu.VMEM((_B, 128), f32),
        pltpu.SemaphoreType.DMA,
        pltpu.SemaphoreType.DMA,
        pltpu.SemaphoreType.DMA,
        pltpu.SemaphoreType.DMA,
        pltpu.SemaphoreType.DMA,
        pltpu.SemaphoreType.DMA,
        pltpu.VMEM_SHARED((_NPAD, 128), f32),
    ]
    mesh = plsc.VectorSubcoreMesh(core_axis_name="c", subcore_axis_name="s")

    def body(z2, edges, t0, t1, a0, a1,
             ib0, ib1, ra, rb, is0, is1, ga, gb, sa, sb, acc):
        c = lax.axis_index("c")
        t = lax.axis_index("s")
        r0 = t * _RPT
        idx0 = pltpu.async_copy(edges.at[t].at[pl.ds(0, _G)], ib0, is0)
        pltpu.sync_copy(z2.at[pl.ds(r0, _RPT)], acc.at[pl.ds(r0, _RPT)])
        plsc.subcore_barrier()
        for cc in range(2):
            tbl = (t0, t1)[cc]
            out = (a0, a1)[cc]

            @pl.when(c == cc)
            def _():
                _run_groups(edges.at[t], tbl, acc, (ib0, ib1), (is0, is1),
                            (ra, rb), (ga, gb), (sa, sb), _G, _CH // _G,
                            idx0=idx0)
                plsc.subcore_barrier()
                pltpu.sync_copy(acc.at[pl.ds(r0, _RPT)],
                                out.at[pl.ds(r0, _RPT)])

    return pl.kernel(body, out_type=outs, mesh=mesh, scratch_types=scratch)


_BN = 1000
_full = lambda shape: pl.BlockSpec(shape, lambda i: (0, 0))
_rows = lambda w: pl.BlockSpec((_BN, w), lambda i: (i, 0))


def _make_tc_r0():
    f32 = jnp.float32

    def body(xf, wr, b, r_o):
        r_o[...] = jnp.dot(xf[...], wr[...], preferred_element_type=f32) + b[...]

    return pl.pallas_call(
        body,
        grid=(_N // _BN,),
        in_specs=[_rows(128), _full((128, _H)), _full((1, _H))],
        out_specs=[_rows(_H)],
        out_shape=[jax.ShapeDtypeStruct((_N, _H), f32)],
    )


def _make_tc_r():
    f32 = jnp.float32
    dh = 128

    def body(h0, h1, wr, b, r_o):
        r_o[...] = (jnp.dot(h0[...], wr[0:dh, :], preferred_element_type=f32)
                    + jnp.dot(h1[...], wr[dh:2 * dh, :],
                              preferred_element_type=f32) + b[...])

    return pl.pallas_call(
        body,
        grid=(_N // _BN,),
        in_specs=[_rows(dh), _rows(dh), _full((2 * dh, _H)), _full((1, _H))],
        out_specs=[_rows(_H)],
        out_shape=[jax.ShapeDtypeStruct((_N, _H), f32)],
    )


def _make_tc_layer0():
    f32 = jnp.float32
    hh = _H // 2

    def body(p0, p1, r, d0, d1, wl, o0, o1, sc_o):
        scale = 1.0 / jnp.maximum(d0[...] + d1[...], 1.0)
        acc = jnp.dot((p0[...] + p1[...]) * scale, wl[...],
                      preferred_element_type=f32) + r[...]
        acc = jnp.maximum(acc, 0.0)
        o0[...] = acc[:, :hh]
        o1[...] = acc[:, hh:]
        sc_o[...] = scale

    return pl.pallas_call(
        body,
        grid=(_N // _BN,),
        in_specs=[_rows(128), _rows(128), _rows(_H), _rows(1), _rows(1),
                  _full((128, _H))],
        out_specs=[_rows(hh), _rows(hh), _rows(1)],
        out_shape=[jax.ShapeDtypeStruct((_N, hh), f32),
                   jax.ShapeDtypeStruct((_N, hh), f32),
                   jax.ShapeDtypeStruct((_N, 1), f32)],
    )


def _make_tc_layer(relu):
    f32 = jnp.float32
    dh = 128
    hh = _H // 2

    def body(a0, a1, r, sc, wl, o0, o1):
        scale = sc[...]
        acc = (jnp.dot(a0[...] * scale, wl[0:dh, :], preferred_element_type=f32)
               + jnp.dot(a1[...] * scale, wl[dh:2 * dh, :],
                         preferred_element_type=f32)
               + r[...])
        if relu:
            acc = jnp.maximum(acc, 0.0)
        o0[...] = acc[:, :hh]
        o1[...] = acc[:, hh:]

    return pl.pallas_call(
        body,
        grid=(_N // _BN,),
        in_specs=[_rows(dh), _rows(dh), _rows(_H), _rows(1),
                  _full((2 * dh, _H))],
        out_specs=[_rows(hh), _rows(hh)],
        out_shape=[jax.ShapeDtypeStruct((_N, hh), f32),
                   jax.ShapeDtypeStruct((_N, hh), f32)],
    )


def _make_tc_final():
    f32 = jnp.float32
    dh = 128

    def body(a0, a1, r, sc, wl,
             ws1, bs1, ws2, bs2, wn1, bn1, wn2, bn2, seg_o, nud_o):
        scale = sc[...]
        h = (jnp.dot(a0[...] * scale, wl[0:dh, :], preferred_element_type=f32)
             + jnp.dot(a1[...] * scale, wl[dh:2 * dh, :],
                       preferred_element_type=f32)
             + r[...])
        sh = jnp.maximum(jnp.dot(h, ws1[...], preferred_element_type=f32)
                         + bs1[...], 0.0)
        seg_o[...] = jnp.dot(sh, ws2[...], preferred_element_type=f32) + bs2[...]
        nh = jnp.maximum(jnp.dot(h, wn1[...], preferred_element_type=f32)
                         + bn1[...], 0.0)
        nud_o[...] = jnp.dot(nh, wn2[...], preferred_element_type=f32) + bn2[...]

    return pl.pallas_call(
        body,
        grid=(_N // _BN,),
        in_specs=[_rows(dh), _rows(dh), _rows(_H), _rows(1),
                  _full((2 * dh, _H)),
                  _full((_H, _H)), _full((1, _H)), _full((_H, 6)), _full((1, 6)),
                  _full((_H, _H)), _full((1, _H)), _full((_H, 5)), _full((1, 5))],
        out_specs=[_rows(6), _rows(5)],
        out_shape=[jax.ShapeDtypeStruct((_N, 6), f32),
                   jax.ShapeDtypeStruct((_N, 5), f32)],
    )


def kernel(x, edge_index, W0_l, W0_r, b0, W1_l, W1_r, b1, W2_l, W2_r, b2,
           Ws1, bs1, Ws2, bs2, Wn1, bn1, Wn2, bn2):
    f32 = jnp.float32
    src = edge_index[0]
    dst = edge_index[1]
    srcs0 = jnp.pad(src.reshape(2 * _TILES, _EPT0),
                    ((0, 0), (0, _CH0 * _B - _EPT0))).reshape(
                        2 * _TILES, _CH0, 1, _B)
    dsts0 = jnp.pad(dst.reshape(2 * _TILES, _EPT0),
                    ((0, 0), (0, _CH0 * _B - _EPT0)),
                    constant_values=_N).reshape(2 * _TILES, _CH0, 1, _B)
    edges0 = jnp.concatenate([srcs0, dsts0], axis=2)
    srcs = jnp.pad(src.reshape(_TILES, _EPT),
                   ((0, 0), (0, _CH * _B - _EPT))).reshape(_TILES, _CH, 1, _B)
    dsts = jnp.pad(dst.reshape(_TILES, _EPT),
                   ((0, 0), (0, _CH * _B - _EPT)),
                   constant_values=_N).reshape(_TILES, _CH, 1, _B)
    edges = jnp.concatenate([srcs, dsts], axis=2)
    z128 = jnp.zeros((_NPAD, 128), f32)
    z1d = jnp.zeros((_NPAD,), f32)

    p0, p1, degp = _make_sc_agg0()(z128, z1d, edges0, x)
    (r,) = _make_tc_r0()(x, W0_r, b0.reshape(1, -1))
    d0 = degp[:_TILES].reshape(-1)[:_N].reshape(_N, 1)
    d1 = degp[_TILES:].reshape(-1)[:_N].reshape(_N, 1)

    h0, h1, scale = _make_tc_layer0()(p0, p1, r, d0, d1, W0_l)

    sc_mid = _make_sc_agg()
    tc_r = _make_tc_r()
    a0p, a1p = sc_mid(z128, edges, h0, h1)
    (r,) = tc_r(h0, h1, W1_r, b1.reshape(1, -1))
    h0, h1 = _make_tc_layer(True)(a0p, a1p, r, scale, W1_l)

    a0p, a1p = sc_mid(z128, edges, h0, h1)
    (r,) = tc_r(h0, h1, W2_r, b2.reshape(1, -1))
    seg, nud = _make_tc_final()(
        a0p, a1p, r, scale, W2_l,
        Ws1, bs1.reshape(1, -1), Ws2, bs2.reshape(1, -1),
        Wn1, bn1.reshape(1, -1), Wn2, bn2.reshape(1, -1))
    return (seg, nud)
